# baseline (speedup 1.0000x reference)
import functools

import jax
import jax.numpy as jnp
from jax import lax
from jax.experimental import pallas as pl
from jax.experimental.pallas import tpu as pltpu
from jax.experimental.pallas import tpu_sc as plsc

BS, SEQ, N, E, D, H, NC, VOCAB = 8, 50, 200, 200, 64, 128, 110, 10000
G = BS * SEQ
F_IN = 2 + NC + 1 + D

_NW = 32
_CHUNK = 128
_NCHUNK = 20
_ROWS_PAD = _NW * _NCHUNK * _CHUNK


@functools.lru_cache(maxsize=1)
def _make_sc_gather():
    @functools.partial(
        pl.kernel,
        out_type=jax.ShapeDtypeStruct((_ROWS_PAD, 128), jnp.float32),
        mesh=plsc.VectorSubcoreMesh(core_axis_name="c", subcore_axis_name="s",
                                    num_cores=2),
        scratch_types=[
            pltpu.VMEM((_NCHUNK, _CHUNK), jnp.int32),

            pltpu.VMEM((_CHUNK, 128), jnp.float32),
            pltpu.VMEM((_CHUNK, 128), jnp.float32),
            pltpu.SemaphoreType.DMA,
            pltpu.SemaphoreType.DMA,
        ],
    )
    def _sc_gather(idx_hbm, table_hbm, out_hbm, idx_v, buf0, buf1, sem0, sem1):
        wid = lax.axis_index("s") * 2 + lax.axis_index("c")
        base = wid * (_NCHUNK * _CHUNK)
        pltpu.sync_copy(idx_hbm.at[wid], idx_v)
        bufs = (buf0, buf1)
        sems = (sem0, sem1)
        copies = [None, None]
        for j in range(_NCHUNK):
            b = j % 2
            copies[b] = pltpu.async_copy(table_hbm.at[idx_v.at[j]], bufs[b], sems[b])
            if j > 0:
                pb = (j - 1) % 2
                copies[pb].wait()
                pltpu.sync_copy(bufs[pb],
                                out_hbm.at[pl.ds(base + (j - 1) * _CHUNK, _CHUNK)])
        last = (_NCHUNK - 1) % 2
        copies[last].wait()
        pltpu.sync_copy(bufs[last],
                        out_hbm.at[pl.ds(base + (_NCHUNK - 1) * _CHUNK, _CHUNK)])

    return _sc_gather


def _gather_rows(table, idx_flat):
    table128 = jnp.concatenate(
        [table, jnp.zeros((table.shape[0], 128 - D), jnp.float32)], axis=1)
    idx_pad = jnp.zeros((_ROWS_PAD,), jnp.int32).at[: idx_flat.shape[0]].set(idx_flat)
    idx3d = idx_pad.reshape(_NW, _NCHUNK, _CHUNK)
    out = _make_sc_gather()(idx3d, table128)
    return out[: idx_flat.shape[0]]


_GB = 4


def _ggnn_body(x_ref, e_ref, et_ref, eet_ref, Wbig_ref, gb_ref, WhhTn_ref,
               bhhn_ref, gateW_ref, gateb_ref, out_ref):
    eet = eet_ref[...]
    et_mean = jnp.mean(eet, axis=1, keepdims=True)

    io_ne = lax.broadcasted_iota(jnp.int32, (N, E), 0)
    io8 = lax.broadcasted_iota(jnp.int32, (8, E), 0)

    Ms = []
    for b in range(_GB):
        e = e_ref[b]
        src_row = e[0:1, :]
        dst_row = e[1:2, :]
        et_row = et_ref[b]
        ST = (io_ne == src_row).astype(jnp.bfloat16)
        DT = (io_ne == dst_row).astype(jnp.float32)
        etoh = (io8 == et_row).astype(jnp.float32)
        ew_row = jnp.sum(etoh * et_mean, axis=0, keepdims=True)
        Ms.append(lax.dot_general((DT * ew_row).astype(jnp.bfloat16), ST,
                                  (((1,), (1,)), ((), ())),
                                  preferred_element_type=jnp.float32)
                  .astype(jnp.bfloat16))

    H0 = x_ref[...][:, :, :D].reshape(_GB * N, D)
    gb = gb_ref[...]
    WhhTn = WhhTn_ref[...]
    bhhn = bhhn_ref[...]

    h = H0
    for l in range(4):
        hb16 = h.astype(jnp.bfloat16)
        Mh = jnp.concatenate(
            [jnp.dot(Ms[b], hb16[b * N:(b + 1) * N, :],
                     preferred_element_type=jnp.float32) for b in range(_GB)],
            axis=0)
        C = jnp.concatenate([Mh.astype(jnp.bfloat16), hb16], axis=1)
        g = jnp.dot(C, Wbig_ref[l], preferred_element_type=jnp.float32) + gb
        r = jax.nn.sigmoid(g[:, :D])
        z = jax.nn.sigmoid(g[:, D:2 * D])
        i_n = g[:, 2 * D:]
        h_n = jnp.dot(hb16, WhhTn, preferred_element_type=jnp.float32) + bhhn
        nn_ = jnp.tanh(i_n + r * h_n)
        h = (1.0 - z) * nn_ + z * h

    gate = jax.nn.sigmoid(
        jnp.sum(h * gateW_ref[...], axis=1, keepdims=True) + gateb_ref[...])
    for b in range(_GB):
        gb_slice = gate[b * N:(b + 1) * N, :]
        hb = h[b * N:(b + 1) * N, :]
        m0 = jnp.max(gb_slice, axis=0, keepdims=True)
        ex = jnp.exp(gb_slice - m0)
        att = ex / jnp.sum(ex, axis=0, keepdims=True)
        out_ref[b] = jnp.sum(att * hb, axis=0, keepdims=True)


def _ggnn_pool_all(x, eidx, etype, eet, Wbig, gbias, WhhTn, bhhn, gateW, gateb):
    return pl.pallas_call(
        _ggnn_body,
        grid=(G // _GB,),
        in_specs=[
            pl.BlockSpec((_GB, N, 128), lambda i: (i, 0, 0)),
            pl.BlockSpec((_GB, 2, E), lambda i: (i, 0, 0)),
            pl.BlockSpec((_GB, 1, E), lambda i: (i, 0, 0)),
            pl.BlockSpec((8, D), lambda i: (0, 0)),
            pl.BlockSpec((4, 2 * D, 3 * D), lambda i: (0, 0, 0)),
            pl.BlockSpec((1, 3 * D), lambda i: (0, 0)),
            pl.BlockSpec((D, D), lambda i: (0, 0)),
            pl.BlockSpec((1, D), lambda i: (0, 0)),
            pl.BlockSpec((1, D), lambda i: (0, 0)),
            pl.BlockSpec((1, 1), lambda i: (0, 0)),
        ],
        out_specs=pl.BlockSpec((_GB, 1, D), lambda i: (i, 0, 0)),
        out_shape=jax.ShapeDtypeStruct((G, 1, D), jnp.float32),
    )(x, eidx, etype, eet, Wbig, gbias, WhhTn, bhhn, gateW, gateb)


def _lstm_body(x_ref, tc_ref, res_ref, WihT_ref, WhhT_ref, bih_ref, bhh_ref,
               predWT_ref, predb_ref, loss_ref, fp_ref):
    WihT = WihT_ref[...]
    WhhT = WhhT_ref[...]
    b = bih_ref[...] + bhh_ref[...]
    predWT = predWT_ref[...]
    predb = predb_ref[...]

    def step(t, carry):
        h, c, ls, cnt = carry
        xt = x_ref[t]
        g = (jnp.dot(xt, WihT, preferred_element_type=jnp.float32)
             + jnp.dot(h, WhhT, preferred_element_type=jnp.float32) + b)
        i = jax.nn.sigmoid(g[:, :H])
        f = jax.nn.sigmoid(g[:, H:2 * H])
        gg = jnp.tanh(g[:, 2 * H:3 * H])
        o = jax.nn.sigmoid(g[:, 3 * H:])
        c = f * c + i * gg
        h = o * jnp.tanh(c)
        pred = jnp.dot(h, predWT, preferred_element_type=jnp.float32) + predb
        tc = tc_ref[t]
        p1 = jnp.sum(pred * tc, axis=1, keepdims=True)
        numc = jnp.sum(tc, axis=1, keepdims=True)
        mask = numc > 0.0
        safe = jnp.where(mask, numc, 1.0)
        fp = p1 / safe
        ft = res_ref[t]
        bce = jnp.maximum(fp, 0.0) - fp * ft + jnp.log1p(jnp.exp(-jnp.abs(fp)))
        ls = ls + jnp.sum(jnp.where(mask, bce, 0.0), keepdims=True).reshape(1, 1)
        cnt = cnt + jnp.sum(mask.astype(jnp.float32), keepdims=True).reshape(1, 1)
        fp_ref[t] = jax.nn.sigmoid(fp)
        return h, c, ls, cnt

    h0 = jnp.zeros((BS, H), jnp.float32)
    c0 = jnp.zeros((BS, H), jnp.float32)
    z11 = jnp.zeros((1, 1), jnp.float32)
    _, _, ls, cnt = lax.fori_loop(0, SEQ, step, (h0, c0, z11, z11))
    loss_ref[...] = ls / jnp.maximum(cnt, 1.0)


def _lstm_head(xseq, tcT, resT, WihT, WhhT, bih, bhh, predWT, predb):
    return pl.pallas_call(
        _lstm_body,
        out_shape=(
            jax.ShapeDtypeStruct((1, 1), jnp.float32),
            jax.ShapeDtypeStruct((SEQ, BS, 1), jnp.float32),
        ),
    )(xseq, tcT, resT, WihT, WhhT, bih, bhh, predWT, predb)


def kernel(p_id, c_id, node_id, edge, edge_type, target_c, result, c_embed,
           cur_result, node_embed_table, edge_embed_table, ggnn_W, gru_W_ih,
           gru_W_hh, gru_b_ih, gru_b_hh, gate_W, gate_b, lstm_W_ih, lstm_W_hh,
           lstm_b_ih, lstm_b_hh, pred_W, pred_b):
    table = node_embed_table.astype(jnp.float32)
    idx_flat = node_id.reshape(-1).astype(jnp.int32)
    x = _gather_rows(table, idx_flat).reshape(G, N, 128)

    eidx = edge.reshape(G, 2, E).astype(jnp.int32)
    etype = edge_type.reshape(G, 1, E).astype(jnp.int32)

    WihT = gru_W_ih.T.astype(jnp.float32)
    WhhT = gru_W_hh.T.astype(jnp.float32)
    ggnnW = ggnn_W.astype(jnp.float32)
    top = jnp.einsum('lij,jk->lik', ggnnW, WihT)
    bot = jnp.concatenate([WhhT[:, :2 * D], jnp.zeros((D, D), jnp.float32)], axis=1)
    Wbig = jnp.concatenate([top, jnp.broadcast_to(bot[None], (4, D, 3 * D))],
                           axis=1).astype(jnp.bfloat16)
    gbias = (gru_b_ih.astype(jnp.float32)
             + jnp.concatenate([gru_b_hh[:2 * D].astype(jnp.float32),
                                jnp.zeros((D,), jnp.float32)])).reshape(1, 3 * D)
    WhhTn = WhhT[:, 2 * D:].astype(jnp.bfloat16)
    bhhn = gru_b_hh[2 * D:].astype(jnp.float32).reshape(1, D)

    codevec = _ggnn_pool_all(
        x, eidx, etype,
        edge_embed_table.astype(jnp.float32),
        Wbig, gbias, WhhTn, bhhn,
        gate_W.reshape(1, D).astype(jnp.float32),
        gate_b.reshape(1, 1).astype(jnp.float32),
    ).reshape(BS, SEQ, D)

    lstm_in = jnp.concatenate([c_embed, codevec, cur_result], axis=2)
    xseq = lstm_in.transpose(1, 0, 2).astype(jnp.float32)
    tcT = target_c.transpose(1, 0, 2).astype(jnp.float32)
    resT = result.reshape(BS, SEQ, 1).transpose(1, 0, 2).astype(jnp.float32)

    loss2d, fp_out = _lstm_head(
        xseq, tcT, resT,
        lstm_W_ih.T.astype(jnp.float32),
        lstm_W_hh.T.astype(jnp.float32),
        lstm_b_ih.reshape(1, 4 * H).astype(jnp.float32),
        lstm_b_hh.reshape(1, 4 * H).astype(jnp.float32),
        pred_W.T.astype(jnp.float32),
        pred_b.reshape(1, NC + 1).astype(jnp.float32),
    )

    loss = loss2d[0, 0]
    fp_sig = fp_out.reshape(SEQ, BS).transpose(1, 0).reshape(BS * SEQ)
    ft = result.reshape(-1)
    return loss, fp_sig, ft


# speedup vs baseline: 1.1149x; 1.1149x over previous
import functools

import jax
import jax.numpy as jnp
from jax import lax
from jax.experimental import pallas as pl
from jax.experimental.pallas import tpu as pltpu
from jax.experimental.pallas import tpu_sc as plsc

BS, SEQ, N, E, D, H, NC, VOCAB = 8, 50, 200, 200, 64, 128, 110, 10000
G = BS * SEQ
F_IN = 2 + NC + 1 + D

_NW = 32
_CHUNK = 128
_NCHUNK = 20
_ROWS_PAD = _NW * _NCHUNK * _CHUNK


@functools.lru_cache(maxsize=1)
def _make_sc_gather():
    @functools.partial(
        pl.kernel,
        out_type=jax.ShapeDtypeStruct((_ROWS_PAD, 128), jnp.float32),
        mesh=plsc.VectorSubcoreMesh(core_axis_name="c", subcore_axis_name="s",
                                    num_cores=2),
        scratch_types=[
            pltpu.VMEM((_NCHUNK, _CHUNK), jnp.int32),

            pltpu.VMEM((_CHUNK, 128), jnp.float32),
            pltpu.VMEM((_CHUNK, 128), jnp.float32),
            pltpu.SemaphoreType.DMA,
            pltpu.SemaphoreType.DMA,
        ],
    )
    def _sc_gather(idx_hbm, table_hbm, out_hbm, idx_v, buf0, buf1, sem0, sem1):
        wid = lax.axis_index("s") * 2 + lax.axis_index("c")
        base = wid * (_NCHUNK * _CHUNK)
        pltpu.sync_copy(idx_hbm.at[wid], idx_v)
        bufs = (buf0, buf1)
        sems = (sem0, sem1)
        copies = [None, None]
        for j in range(_NCHUNK):
            b = j % 2
            copies[b] = pltpu.async_copy(table_hbm.at[idx_v.at[j]], bufs[b], sems[b])
            if j > 0:
                pb = (j - 1) % 2
                copies[pb].wait()
                pltpu.sync_copy(bufs[pb],
                                out_hbm.at[pl.ds(base + (j - 1) * _CHUNK, _CHUNK)])
        last = (_NCHUNK - 1) % 2
        copies[last].wait()
        pltpu.sync_copy(bufs[last],
                        out_hbm.at[pl.ds(base + (_NCHUNK - 1) * _CHUNK, _CHUNK)])

    return _sc_gather


def _gather_rows(table, idx_flat):
    table128 = jnp.concatenate(
        [table, jnp.zeros((table.shape[0], 128 - D), jnp.float32)], axis=1)
    idx_pad = jnp.zeros((_ROWS_PAD,), jnp.int32).at[: idx_flat.shape[0]].set(idx_flat)
    idx3d = idx_pad.reshape(_NW, _NCHUNK, _CHUNK)
    out = _make_sc_gather()(idx3d, table128)
    return out[: idx_flat.shape[0]]


_GB = 8


def _ggnn_body(x_ref, e_ref, et_ref, eet_ref, Wbig_ref, gb_ref, WhhTn_ref,
               bhhn_ref, gateW_ref, gateb_ref, out_ref):
    eet = eet_ref[...]
    et_mean = jnp.mean(eet, axis=1, keepdims=True)

    io_ne = lax.broadcasted_iota(jnp.int32, (N, E), 0)
    io8 = lax.broadcasted_iota(jnp.int32, (8, E), 0)

    Ms = []
    for b in range(_GB):
        e = e_ref[b]
        src_row = e[0:1, :]
        dst_row = e[1:2, :]
        et_row = et_ref[b]
        ST = (io_ne == src_row).astype(jnp.float32)
        DT = (io_ne == dst_row).astype(jnp.float32)
        etoh = (io8 == et_row).astype(jnp.float32)
        ew_row = jnp.sum(etoh * et_mean, axis=0, keepdims=True)
        Ms.append(lax.dot_general(DT * ew_row, ST, (((1,), (1,)), ((), ())),
                                  preferred_element_type=jnp.float32))

    H0 = x_ref[...][:, :, :D].reshape(_GB * N, D)
    gb = gb_ref[...]
    WhhTn = WhhTn_ref[...]
    bhhn = bhhn_ref[...]

    h = H0
    for l in range(4):
        Mh = jnp.concatenate(
            [jnp.dot(Ms[b], h[b * N:(b + 1) * N, :],
                     preferred_element_type=jnp.float32) for b in range(_GB)],
            axis=0)
        C = jnp.concatenate([Mh, h], axis=1)
        g = jnp.dot(C, Wbig_ref[l], preferred_element_type=jnp.float32) + gb
        r = jax.nn.sigmoid(g[:, :D])
        z = jax.nn.sigmoid(g[:, D:2 * D])
        i_n = g[:, 2 * D:]
        h_n = jnp.dot(h, WhhTn, preferred_element_type=jnp.float32) + bhhn
        nn_ = jnp.tanh(i_n + r * h_n)
        h = (1.0 - z) * nn_ + z * h

    gate = jax.nn.sigmoid(
        jnp.sum(h * gateW_ref[...], axis=1, keepdims=True) + gateb_ref[...])
    for b in range(_GB):
        gb_slice = gate[b * N:(b + 1) * N, :]
        hb = h[b * N:(b + 1) * N, :]
        m0 = jnp.max(gb_slice, axis=0, keepdims=True)
        ex = jnp.exp(gb_slice - m0)
        att = ex / jnp.sum(ex, axis=0, keepdims=True)
        out_ref[b] = jnp.sum(att * hb, axis=0, keepdims=True)


def _ggnn_pool_all(x, eidx, etype, eet, Wbig, gbias, WhhTn, bhhn, gateW, gateb):
    return pl.pallas_call(
        _ggnn_body,
        grid=(G // _GB,),
        in_specs=[
            pl.BlockSpec((_GB, N, 128), lambda i: (i, 0, 0)),
            pl.BlockSpec((_GB, 2, E), lambda i: (i, 0, 0)),
            pl.BlockSpec((_GB, 1, E), lambda i: (i, 0, 0)),
            pl.BlockSpec((8, D), lambda i: (0, 0)),
            pl.BlockSpec((4, 2 * D, 3 * D), lambda i: (0, 0, 0)),
            pl.BlockSpec((1, 3 * D), lambda i: (0, 0)),
            pl.BlockSpec((D, D), lambda i: (0, 0)),
            pl.BlockSpec((1, D), lambda i: (0, 0)),
            pl.BlockSpec((1, D), lambda i: (0, 0)),
            pl.BlockSpec((1, 1), lambda i: (0, 0)),
        ],
        out_specs=pl.BlockSpec((_GB, 1, D), lambda i: (i, 0, 0)),
        out_shape=jax.ShapeDtypeStruct((G, 1, D), jnp.float32),
    )(x, eidx, etype, eet, Wbig, gbias, WhhTn, bhhn, gateW, gateb)


def _lstm_body(x_ref, tc_ref, res_ref, WihT_ref, WhhT_ref, bih_ref, bhh_ref,
               predWT_ref, predb_ref, loss_ref, fp_ref):
    WihT = WihT_ref[...]
    WhhT = WhhT_ref[...]
    b = bih_ref[...] + bhh_ref[...]
    predWT = predWT_ref[...]
    predb = predb_ref[...]

    def step(t, carry):
        h, c, ls, cnt = carry
        xt = x_ref[t]
        g = (jnp.dot(xt, WihT, preferred_element_type=jnp.float32)
             + jnp.dot(h, WhhT, preferred_element_type=jnp.float32) + b)
        i = jax.nn.sigmoid(g[:, :H])
        f = jax.nn.sigmoid(g[:, H:2 * H])
        gg = jnp.tanh(g[:, 2 * H:3 * H])
        o = jax.nn.sigmoid(g[:, 3 * H:])
        c = f * c + i * gg
        h = o * jnp.tanh(c)
        pred = jnp.dot(h, predWT, preferred_element_type=jnp.float32) + predb
        tc = tc_ref[t]
        p1 = jnp.sum(pred * tc, axis=1, keepdims=True)
        numc = jnp.sum(tc, axis=1, keepdims=True)
        mask = numc > 0.0
        safe = jnp.where(mask, numc, 1.0)
        fp = p1 / safe
        ft = res_ref[t]
        bce = jnp.maximum(fp, 0.0) - fp * ft + jnp.log1p(jnp.exp(-jnp.abs(fp)))
        ls = ls + jnp.sum(jnp.where(mask, bce, 0.0), keepdims=True).reshape(1, 1)
        cnt = cnt + jnp.sum(mask.astype(jnp.float32), keepdims=True).reshape(1, 1)
        fp_ref[t] = jax.nn.sigmoid(fp)
        return h, c, ls, cnt

    h0 = jnp.zeros((BS, H), jnp.float32)
    c0 = jnp.zeros((BS, H), jnp.float32)
    z11 = jnp.zeros((1, 1), jnp.float32)
    _, _, ls, cnt = lax.fori_loop(0, SEQ, step, (h0, c0, z11, z11))
    loss_ref[...] = ls / jnp.maximum(cnt, 1.0)


def _lstm_head(xseq, tcT, resT, WihT, WhhT, bih, bhh, predWT, predb):
    return pl.pallas_call(
        _lstm_body,
        out_shape=(
            jax.ShapeDtypeStruct((1, 1), jnp.float32),
            jax.ShapeDtypeStruct((SEQ, BS, 1), jnp.float32),
        ),
    )(xseq, tcT, resT, WihT, WhhT, bih, bhh, predWT, predb)


def kernel(p_id, c_id, node_id, edge, edge_type, target_c, result, c_embed,
           cur_result, node_embed_table, edge_embed_table, ggnn_W, gru_W_ih,
           gru_W_hh, gru_b_ih, gru_b_hh, gate_W, gate_b, lstm_W_ih, lstm_W_hh,
           lstm_b_ih, lstm_b_hh, pred_W, pred_b):
    table = node_embed_table.astype(jnp.float32)
    idx_flat = node_id.reshape(-1).astype(jnp.int32)
    x = _gather_rows(table, idx_flat).reshape(G, N, 128)

    eidx = edge.reshape(G, 2, E).astype(jnp.int32)
    etype = edge_type.reshape(G, 1, E).astype(jnp.int32)

    WihT = gru_W_ih.T.astype(jnp.float32)
    WhhT = gru_W_hh.T.astype(jnp.float32)
    ggnnW = ggnn_W.astype(jnp.float32)
    top = jnp.einsum('lij,jk->lik', ggnnW, WihT)
    bot = jnp.concatenate([WhhT[:, :2 * D], jnp.zeros((D, D), jnp.float32)], axis=1)
    Wbig = jnp.concatenate([top, jnp.broadcast_to(bot[None], (4, D, 3 * D))],
                           axis=1)
    gbias = (gru_b_ih.astype(jnp.float32)
             + jnp.concatenate([gru_b_hh[:2 * D].astype(jnp.float32),
                                jnp.zeros((D,), jnp.float32)])).reshape(1, 3 * D)
    WhhTn = WhhT[:, 2 * D:]
    bhhn = gru_b_hh[2 * D:].astype(jnp.float32).reshape(1, D)

    codevec = _ggnn_pool_all(
        x, eidx, etype,
        edge_embed_table.astype(jnp.float32),
        Wbig, gbias, WhhTn, bhhn,
        gate_W.reshape(1, D).astype(jnp.float32),
        gate_b.reshape(1, 1).astype(jnp.float32),
    ).reshape(BS, SEQ, D)

    lstm_in = jnp.concatenate([c_embed, codevec, cur_result], axis=2)
    xseq = lstm_in.transpose(1, 0, 2).astype(jnp.float32)
    tcT = target_c.transpose(1, 0, 2).astype(jnp.float32)
    resT = result.reshape(BS, SEQ, 1).transpose(1, 0, 2).astype(jnp.float32)

    loss2d, fp_out = _lstm_head(
        xseq, tcT, resT,
        lstm_W_ih.T.astype(jnp.float32),
        lstm_W_hh.T.astype(jnp.float32),
        lstm_b_ih.reshape(1, 4 * H).astype(jnp.float32),
        lstm_b_hh.reshape(1, 4 * H).astype(jnp.float32),
        pred_W.T.astype(jnp.float32),
        pred_b.reshape(1, NC + 1).astype(jnp.float32),
    )

    loss = loss2d[0, 0]
    fp_sig = fp_out.reshape(SEQ, BS).transpose(1, 0).reshape(BS * SEQ)
    ft = result.reshape(-1)
    return loss, fp_sig, ft
